# baseline (speedup 1.0000x reference)
import functools

import jax
import jax.numpy as jnp
import numpy as np
from jax import lax
from jax.experimental import pallas as pl
from jax.experimental.pallas import tpu as pltpu
from jax.experimental.pallas import tpu_sc as plsc

VOCAB = 1000000
D = 64
BATCH = 4096
SEQ = 200
N = BATCH * SEQ
NC, NS = 2, 16
NW = NC * NS
PER_W = N // NW
CHUNKS = PER_W // SEQ
SPLIT = 128
LANES = 16
VREGS_PER_ROW = D // LANES


def _positional_encoding() -> np.ndarray:
    position = np.arange(0, SEQ, dtype=np.float32)[:, None]
    div_term = np.exp(np.arange(0, D, 2, dtype=np.float32) * (-np.log(10000.0) / D))
    pe = np.zeros((SEQ, D), dtype=np.float32)
    pe[:, 0::2] = np.sin(position * div_term)
    pe[:, 1::2] = np.cos(position * div_term)
    return pe


_PE = _positional_encoding()


def _body(table_hbm, idx_hbm, pe_hbm, out_hbm,
          idx_v, pe_v, gbuf0, gbuf1, obuf0, obuf1,
          gsem0, gsem1, osem0, osem1):
    wid = lax.axis_index("s") * NC + lax.axis_index("c")
    base = pl.multiple_of(wid * PER_W, 8)

    pltpu.sync_copy(idx_hbm.at[pl.ds(base, PER_W)], idx_v)
    pltpu.sync_copy(pe_hbm, pe_v)

    gbufs = (gbuf0, gbuf1)
    obufs = (obuf0, obuf1)
    gsems = (gsem0, gsem1)
    osems = (osem0, osem1)

    def start_gather(g, b):
        lo = pl.multiple_of(g * SEQ, 8)
        pltpu.async_copy(table_hbm.at[idx_v.at[pl.ds(lo, SPLIT)]],
                         gbufs[b].at[pl.ds(0, SPLIT)], gsems[b])
        pltpu.async_copy(table_hbm.at[idx_v.at[pl.ds(lo + SPLIT, SEQ - SPLIT)]],
                         gbufs[b].at[pl.ds(SPLIT, SEQ - SPLIT)], gsems[b])

    def wait_gather(b):
        pltpu.make_async_copy(table_hbm.at[idx_v.at[pl.ds(0, SPLIT)]],
                              gbufs[b].at[pl.ds(0, SPLIT)], gsems[b]).wait()
        pltpu.make_async_copy(table_hbm.at[idx_v.at[pl.ds(0, SEQ - SPLIT)]],
                              gbufs[b].at[pl.ds(SPLIT, SEQ - SPLIT)], gsems[b]).wait()

    def start_out(g, b):
        pltpu.async_copy(obufs[b], out_hbm.at[pl.ds(base + g * SEQ, SEQ)], osems[b])

    def wait_out(b):
        pltpu.make_async_copy(obufs[b], out_hbm.at[pl.ds(0, SEQ)], osems[b]).wait()

    def add_pe(b):
        gb, ob = gbufs[b], obufs[b]

        def row(r, _):
            for j in range(VREGS_PER_ROW):
                sl = pl.ds(j * LANES, LANES)
                ob[r, sl] = gb[r, sl] + pe_v[r, sl]
            return ()

        lax.fori_loop(0, SEQ, row, (), unroll=2)

    start_gather(0, 0)
    start_gather(1, 1)

    for g in (0, 1):
        b = g % 2
        wait_gather(b)
        add_pe(b)
        start_gather(g + 2, b)
        start_out(g, b)

    def step(g, _):
        b = lax.rem(g, 2)

        def slot(b):
            wait_out(b)
            wait_gather(b)
            add_pe(b)
            start_gather(g + 2, b)
            start_out(g, b)

        @pl.when(b == 0)
        def _():
            slot(0)

        @pl.when(b == 1)
        def _():
            slot(1)

        return ()

    lax.fori_loop(2, CHUNKS - 2, step, ())

    for g in (CHUNKS - 2, CHUNKS - 1):
        b = g % 2
        wait_out(b)
        wait_gather(b)
        add_pe(b)
        start_out(g, b)

    wait_out(0)
    wait_out(1)


@jax.jit
def _run(x_flat, glove_table, pe):
    mesh = plsc.VectorSubcoreMesh(core_axis_name="c", subcore_axis_name="s")
    kern = pl.kernel(
        _body,
        out_type=jax.ShapeDtypeStruct((N, D), jnp.float32),
        mesh=mesh,
        compiler_params=pltpu.CompilerParams(use_tc_tiling_on_sc=False),
        scratch_types=[
            pltpu.VMEM((PER_W,), jnp.int32),
            pltpu.VMEM((SEQ, D), jnp.float32),
            pltpu.VMEM((SEQ, D), jnp.float32),
            pltpu.VMEM((SEQ, D), jnp.float32),
            pltpu.VMEM((SEQ, D), jnp.float32),
            pltpu.VMEM((SEQ, D), jnp.float32),
            pltpu.SemaphoreType.DMA,
            pltpu.SemaphoreType.DMA,
            pltpu.SemaphoreType.DMA,
            pltpu.SemaphoreType.DMA,
        ],
    )
    return kern(glove_table, x_flat, pe)


def kernel(x, glove_table):
    pe = jnp.asarray(_PE)
    out = _run(x.reshape(-1), glove_table, pe)
    return out.reshape(BATCH, SEQ, D)
